# Initial kernel scaffold; baseline (speedup 1.0000x reference)
import jax
import jax.numpy as jnp
from jax.experimental import pallas as pl


def kernel(pred_coordinate, true_coordinate, lddt_mask):
    raise NotImplementedError("write your pallas kernel here")


# speedup vs baseline: 2.2997x; 2.2997x over previous
import jax
import jax.numpy as jnp
from jax.experimental import pallas as pl

_BLK = 128


def _lddt_body(pred_l_ref, pred_m_ref, true_l_ref, true_m_ref, mask_ref, out_ref):
    i = pl.program_id(0)

    @pl.when(i == 0)
    def _init():
        out_ref[...] = jnp.zeros_like(out_ref)

    n_sample = pred_l_ref.shape[0]

    txl = true_l_ref[:, 0:1]
    tyl = true_l_ref[:, 1:2]
    tzl = true_l_ref[:, 2:3]
    txm = true_m_ref[0:1, :]
    tym = true_m_ref[1:2, :]
    tzm = true_m_ref[2:3, :]
    dx = txl - txm
    dy = tyl - tym
    dz = tzl - tzm
    dt = jnp.sqrt(dx * dx + dy * dy + dz * dz)

    maskf = mask_ref[...].astype(jnp.float32)
    out_ref[n_sample : n_sample + 1, 0:1] = out_ref[
        n_sample : n_sample + 1, 0:1
    ] + jnp.sum(maskf).reshape(1, 1)

    for s in range(n_sample):
        pxl = pred_l_ref[s, :, 0:1]
        pyl = pred_l_ref[s, :, 1:2]
        pzl = pred_l_ref[s, :, 2:3]
        pxm = pred_m_ref[s, 0:1, :]
        pym = pred_m_ref[s, 1:2, :]
        pzm = pred_m_ref[s, 2:3, :]
        ex = pxl - pxm
        ey = pyl - pym
        ez = pzl - pzm
        dp = jnp.sqrt(ex * ex + ey * ey + ez * ez)
        err = jnp.abs(dp - dt)
        cnt = (
            (err < 0.5).astype(jnp.float32)
            + (err < 1.0).astype(jnp.float32)
            + (err < 2.0).astype(jnp.float32)
            + (err < 4.0).astype(jnp.float32)
        )
        ssum = jnp.sum(cnt * maskf) * 0.25
        out_ref[s : s + 1, 0:1] = out_ref[s : s + 1, 0:1] + ssum.reshape(1, 1)


def kernel(pred_coordinate, true_coordinate, lddt_mask):
    n_sample, n_atom, _ = pred_coordinate.shape
    pred_m = jnp.transpose(pred_coordinate, (0, 2, 1))
    true_m = true_coordinate.T
    grid = (n_atom // _BLK,)
    out = pl.pallas_call(
        _lddt_body,
        grid=grid,
        in_specs=[
            pl.BlockSpec((n_sample, _BLK, 3), lambda i: (0, i, 0)),
            pl.BlockSpec((n_sample, 3, n_atom), lambda i: (0, 0, 0)),
            pl.BlockSpec((_BLK, 3), lambda i: (i, 0)),
            pl.BlockSpec((3, n_atom), lambda i: (0, 0)),
            pl.BlockSpec((_BLK, n_atom), lambda i: (i, 0)),
        ],
        out_specs=pl.BlockSpec((8, 128), lambda i: (0, 0)),
        out_shape=jax.ShapeDtypeStruct((8, 128), jnp.float32),
    )(pred_coordinate, pred_m, true_coordinate, true_m, lddt_mask)
    return out[:n_sample, 0] / out[n_sample, 0]
